# baseline (speedup 1.0000x reference)
import functools

import jax
import jax.numpy as jnp
from jax import lax
from jax.experimental import pallas as pl
from jax.experimental.pallas import tpu as pltpu
from jax.experimental.pallas import tpu_sc as plsc

BATCH = 16384
VOCAB = 100000
D = 128
OD = D + 1
NC = 2
NS = 16
NW = NC * NS
ROWS_PER_W = BATCH // NW
CHUNK = 128
NCHUNK = ROWS_PER_W // CHUNK
NBUF = 3
LANES = 16


def _sc_body(idx_hbm, expr_hbm, tab_hbm, tok_hbm, out_hbm,
             idx_v, tok_v,
             rows0, rows1, rows2,
             gsem0, gsem1, gsem2, ssem0, ssem1, ssem2, esem):
    rows = (rows0, rows1, rows2)
    gsem = (gsem0, gsem1, gsem2)
    ssem = (ssem0, ssem1, ssem2)

    wid = lax.axis_index("s") * NC + lax.axis_index("c")
    row0 = wid * ROWS_PER_W
    tab = tab_hbm.at[0]

    pltpu.sync_copy(idx_hbm.at[pl.ds(row0, ROWS_PER_W)], idx_v)
    pltpu.sync_copy(tok_hbm, tok_v)

    ecp = pltpu.async_copy(
        expr_hbm.at[pl.ds(row0, ROWS_PER_W)],
        out_hbm.at[pl.ds(row0, ROWS_PER_W), 0, pl.ds(D, 1)],
        esem)

    tok = [tok_v[pl.ds(j * LANES, LANES)] for j in range(D // LANES)]

    def compute_chunk(buf):
        @plsc.parallel_loop(0, CHUNK, unroll=4)
        def _(r):
            for j in range(D // LANES):
                sl = pl.ds(j * LANES, LANES)
                rows[buf][r, sl] = rows[buf][r, sl] + tok[j]

    def gather(c):
        return pltpu.async_copy(
            tab.at[idx_v.at[pl.ds(c * CHUNK, CHUNK)]], rows[c % NBUF],
            gsem[c % NBUF])

    def store(c):
        return pltpu.async_copy(
            rows[c % NBUF],
            out_hbm.at[pl.ds(row0 + c * CHUNK, CHUNK), 0, pl.ds(0, D)],
            ssem[c % NBUF])

    g0 = gather(0)
    g1 = gather(1)
    g2 = gather(2)
    g0.wait()
    compute_chunk(0)
    s0 = store(0)
    g1.wait()
    compute_chunk(1)
    s1 = store(1)
    s0.wait()
    g3 = gather(3)
    g2.wait()
    compute_chunk(2)
    s2 = store(2)
    g3.wait()
    compute_chunk(0)
    s3 = store(3)
    s1.wait()
    s2.wait()
    s3.wait()
    ecp.wait()


@jax.jit
def _pert_embed(idx, expr, pos, tok):
    mesh = plsc.VectorSubcoreMesh(core_axis_name="c", subcore_axis_name="s")
    run = functools.partial(
        pl.kernel, mesh=mesh,
        compiler_params=pltpu.CompilerParams(use_tc_tiling_on_sc=False),
        out_type=jax.ShapeDtypeStruct((BATCH, 1, OD), jnp.float32),
        scratch_types=[
            pltpu.VMEM((ROWS_PER_W,), jnp.int32),
            pltpu.VMEM((D,), jnp.float32),
            pltpu.VMEM((CHUNK, D), jnp.float32),
            pltpu.VMEM((CHUNK, D), jnp.float32),
            pltpu.VMEM((CHUNK, D), jnp.float32),
            pltpu.SemaphoreType.DMA,
            pltpu.SemaphoreType.DMA,
            pltpu.SemaphoreType.DMA,
            pltpu.SemaphoreType.DMA,
            pltpu.SemaphoreType.DMA,
            pltpu.SemaphoreType.DMA,
            pltpu.SemaphoreType.DMA,
        ],
    )(_sc_body)
    return run(idx, expr, pos, tok)


def kernel(pert_index, pert_expression, pos, pert_token):
    return _pert_embed(pert_index.astype(jnp.int32),
                       pert_expression.reshape(BATCH, 1), pos, pert_token)


# speedup vs baseline: 1.4668x; 1.4668x over previous
import functools

import jax
import jax.numpy as jnp
from jax import lax
from jax.experimental import pallas as pl
from jax.experimental.pallas import tpu as pltpu
from jax.experimental.pallas import tpu_sc as plsc

BATCH = 16384
VOCAB = 100000
D = 128
OD = D + 1
NC = 2
NS = 16
NW = NC * NS
ROWS_PER_W = BATCH // NW
CHUNK = 128
NCHUNK = ROWS_PER_W // CHUNK
NBUF = 3
LANES = 16


def _sc_body(idx_hbm, expr_hbm, tab_hbm, tok_hbm, out_hbm,
             idx_v, tok_v,
             rows0, rows1, rows2,
             gsem0, gsem1, gsem2, ssem0, ssem1, ssem2, esem):
    rows = (rows0, rows1, rows2)
    gsem = (gsem0, gsem1, gsem2)
    ssem = (ssem0, ssem1, ssem2)

    wid = lax.axis_index("s") * NC + lax.axis_index("c")
    row0 = wid * ROWS_PER_W
    tab = tab_hbm.at[0]

    pltpu.sync_copy(idx_hbm.at[pl.ds(row0, ROWS_PER_W)], idx_v)
    pltpu.sync_copy(tok_hbm, tok_v)

    ecp = pltpu.async_copy(
        expr_hbm.at[pl.ds(row0, 1)],
        out_hbm.at[pl.ds(row0, 1), 0, pl.ds(D, 1)],
        esem)

    tok = [tok_v[pl.ds(j * LANES, LANES)] for j in range(D // LANES)]

    def compute_chunk(buf):
        @plsc.parallel_loop(0, CHUNK, unroll=4)
        def _(r):
            for j in range(D // LANES):
                sl = pl.ds(j * LANES, LANES)
                rows[buf][r, sl] = rows[buf][r, sl] + tok[j]

    def gather(c):
        return pltpu.async_copy(
            tab.at[idx_v.at[pl.ds(c * CHUNK, CHUNK)]], rows[c % NBUF],
            gsem[c % NBUF])

    def store(c):
        return pltpu.async_copy(
            rows[c % NBUF],
            out_hbm.at[pl.ds(row0 + c * CHUNK, CHUNK), 0, pl.ds(0, D)],
            ssem[c % NBUF])

    g0 = gather(0)
    g1 = gather(1)
    g2 = gather(2)
    g0.wait()
    compute_chunk(0)
    s0 = store(0)
    g1.wait()
    compute_chunk(1)
    s1 = store(1)
    s0.wait()
    g3 = gather(3)
    g2.wait()
    compute_chunk(2)
    s2 = store(2)
    g3.wait()
    compute_chunk(0)
    s3 = store(3)
    s1.wait()
    s2.wait()
    s3.wait()
    ecp.wait()


@jax.jit
def _pert_embed(idx, expr, pos, tok):
    mesh = plsc.VectorSubcoreMesh(core_axis_name="c", subcore_axis_name="s")
    run = functools.partial(
        pl.kernel, mesh=mesh,
        compiler_params=pltpu.CompilerParams(use_tc_tiling_on_sc=False),
        out_type=jax.ShapeDtypeStruct((BATCH, 1, OD), jnp.float32),
        scratch_types=[
            pltpu.VMEM((ROWS_PER_W,), jnp.int32),
            pltpu.VMEM((D,), jnp.float32),
            pltpu.VMEM((CHUNK, D), jnp.float32),
            pltpu.VMEM((CHUNK, D), jnp.float32),
            pltpu.VMEM((CHUNK, D), jnp.float32),
            pltpu.SemaphoreType.DMA,
            pltpu.SemaphoreType.DMA,
            pltpu.SemaphoreType.DMA,
            pltpu.SemaphoreType.DMA,
            pltpu.SemaphoreType.DMA,
            pltpu.SemaphoreType.DMA,
            pltpu.SemaphoreType.DMA,
        ],
    )(_sc_body)
    return run(idx, expr, pos, tok)


def kernel(pert_index, pert_expression, pos, pert_token):
    return _pert_embed(pert_index.astype(jnp.int32),
                       pert_expression.reshape(BATCH, 1), pos, pert_token)


# speedup vs baseline: 1.4781x; 1.0078x over previous
import functools

import jax
import jax.numpy as jnp
from jax import lax
from jax.experimental import pallas as pl
from jax.experimental.pallas import tpu as pltpu
from jax.experimental.pallas import tpu_sc as plsc

BATCH = 16384
VOCAB = 100000
D = 128
OD = D + 1
NC = 2
NS = 16
NW = NC * NS
ROWS_PER_W = BATCH // NW
CHUNK = 128
NCHUNK = ROWS_PER_W // CHUNK
NBUF = 3
LANES = 16


def _sc_body(idx_hbm, expr_hbm, tab_hbm, tok_hbm, out_hbm,
             idx_v, tok_v,
             rows0, rows1, rows2,
             gsem0, gsem1, gsem2, ssem0, ssem1, ssem2, esem):
    rows = (rows0, rows1, rows2)
    gsem = (gsem0, gsem1, gsem2)
    ssem = (ssem0, ssem1, ssem2)

    wid = lax.axis_index("s") * NC + lax.axis_index("c")
    row0 = wid * ROWS_PER_W
    tab = tab_hbm.at[0]

    pltpu.sync_copy(idx_hbm.at[pl.ds(row0, ROWS_PER_W)], idx_v)
    pltpu.sync_copy(tok_hbm, tok_v)

    ecp = pltpu.async_copy(
        expr_hbm.at[pl.ds(row0, 1)],
        out_hbm.at[pl.ds(row0, 1), 0, pl.ds(D, 1)],
        esem)

    tok = [tok_v[pl.ds(j * LANES, LANES)] for j in range(D // LANES)]

    def compute_chunk(buf):
        @plsc.parallel_loop(0, 1, unroll=1)
        def _(r):
            for j in range(1):
                sl = pl.ds(j * LANES, LANES)
                rows[buf][r, sl] = rows[buf][r, sl] + tok[j]

    def gather(c):
        return pltpu.async_copy(
            tab.at[idx_v.at[pl.ds(c * CHUNK, CHUNK)]], rows[c % NBUF],
            gsem[c % NBUF])

    def store(c):
        return pltpu.async_copy(
            rows[c % NBUF],
            out_hbm.at[pl.ds(row0 + c * CHUNK, CHUNK), 0, pl.ds(0, D)],
            ssem[c % NBUF])

    g0 = gather(0)
    g1 = gather(1)
    g2 = gather(2)
    g0.wait()
    compute_chunk(0)
    s0 = store(0)
    g1.wait()
    compute_chunk(1)
    s1 = store(1)
    s0.wait()
    g3 = gather(3)
    g2.wait()
    compute_chunk(2)
    s2 = store(2)
    g3.wait()
    compute_chunk(0)
    s3 = store(3)
    s1.wait()
    s2.wait()
    s3.wait()
    ecp.wait()


@jax.jit
def _pert_embed(idx, expr, pos, tok):
    mesh = plsc.VectorSubcoreMesh(core_axis_name="c", subcore_axis_name="s")
    run = functools.partial(
        pl.kernel, mesh=mesh,
        compiler_params=pltpu.CompilerParams(use_tc_tiling_on_sc=False),
        out_type=jax.ShapeDtypeStruct((BATCH, 1, OD), jnp.float32),
        scratch_types=[
            pltpu.VMEM((ROWS_PER_W,), jnp.int32),
            pltpu.VMEM((D,), jnp.float32),
            pltpu.VMEM((CHUNK, D), jnp.float32),
            pltpu.VMEM((CHUNK, D), jnp.float32),
            pltpu.VMEM((CHUNK, D), jnp.float32),
            pltpu.SemaphoreType.DMA,
            pltpu.SemaphoreType.DMA,
            pltpu.SemaphoreType.DMA,
            pltpu.SemaphoreType.DMA,
            pltpu.SemaphoreType.DMA,
            pltpu.SemaphoreType.DMA,
            pltpu.SemaphoreType.DMA,
        ],
    )(_sc_body)
    return run(idx, expr, pos, tok)


def kernel(pert_index, pert_expression, pos, pert_token):
    return _pert_embed(pert_index.astype(jnp.int32),
                       pert_expression.reshape(BATCH, 1), pos, pert_token)


# speedup vs baseline: 1.5059x; 1.0188x over previous
import functools

import jax
import jax.numpy as jnp
from jax import lax
from jax.experimental import pallas as pl
from jax.experimental.pallas import tpu as pltpu
from jax.experimental.pallas import tpu_sc as plsc

BATCH = 16384
VOCAB = 100000
D = 128
OD = D + 1
NC = 2
NS = 16
NW = NC * NS
ROWS_PER_W = BATCH // NW
CHUNK = 128
NCHUNK = ROWS_PER_W // CHUNK
NBUF = 3
LANES = 16


def _sc_body(idx_hbm, expr_hbm, tab_hbm, tok_hbm, out_hbm,
             idx_v, tok_v,
             rows0, rows1, rows2,
             gsem0, gsem1, gsem2, ssem0, ssem1, ssem2, esem):
    rows = (rows0, rows1, rows2)
    gsem = (gsem0, gsem1, gsem2)
    ssem = (ssem0, ssem1, ssem2)

    wid = lax.axis_index("s") * NC + lax.axis_index("c")
    row0 = wid * ROWS_PER_W
    tab = tab_hbm.at[0]

    pltpu.sync_copy(idx_hbm.at[pl.ds(row0, ROWS_PER_W)], idx_v)
    pltpu.sync_copy(tok_hbm, tok_v)

    ecp = pltpu.async_copy(
        expr_hbm.at[pl.ds(row0, 1)],
        out_hbm.at[pl.ds(row0, 1), 0, pl.ds(D, 1)],
        esem)

    tok = [tok_v[pl.ds(j * LANES, LANES)] for j in range(D // LANES)]

    def compute_chunk(buf):
        @plsc.parallel_loop(0, 1, unroll=1)
        def _(r):
            for j in range(1):
                sl = pl.ds(j * LANES, LANES)
                rows[buf][r, sl] = rows[buf][r, sl] + tok[j]

    def gather(c):
        return pltpu.async_copy(
            tab.at[idx_v.at[pl.ds(c * CHUNK, CHUNK)]], rows[c % NBUF],
            gsem[c % NBUF])

    def store(c):
        return pltpu.async_copy(
            rows[c % NBUF].at[pl.ds(0, 8)],
            out_hbm.at[pl.ds(row0 + c * CHUNK, 8), 0, pl.ds(0, D)],
            ssem[c % NBUF])

    g0 = gather(0)
    g1 = gather(1)
    g2 = gather(2)
    g0.wait()
    compute_chunk(0)
    s0 = store(0)
    g1.wait()
    compute_chunk(1)
    s1 = store(1)
    s0.wait()
    g3 = gather(3)
    g2.wait()
    compute_chunk(2)
    s2 = store(2)
    g3.wait()
    compute_chunk(0)
    s3 = store(3)
    s1.wait()
    s2.wait()
    s3.wait()
    ecp.wait()


@jax.jit
def _pert_embed(idx, expr, pos, tok):
    mesh = plsc.VectorSubcoreMesh(core_axis_name="c", subcore_axis_name="s")
    run = functools.partial(
        pl.kernel, mesh=mesh,
        compiler_params=pltpu.CompilerParams(use_tc_tiling_on_sc=False),
        out_type=jax.ShapeDtypeStruct((BATCH, 1, OD), jnp.float32),
        scratch_types=[
            pltpu.VMEM((ROWS_PER_W,), jnp.int32),
            pltpu.VMEM((D,), jnp.float32),
            pltpu.VMEM((CHUNK, D), jnp.float32),
            pltpu.VMEM((CHUNK, D), jnp.float32),
            pltpu.VMEM((CHUNK, D), jnp.float32),
            pltpu.SemaphoreType.DMA,
            pltpu.SemaphoreType.DMA,
            pltpu.SemaphoreType.DMA,
            pltpu.SemaphoreType.DMA,
            pltpu.SemaphoreType.DMA,
            pltpu.SemaphoreType.DMA,
            pltpu.SemaphoreType.DMA,
        ],
    )(_sc_body)
    return run(idx, expr, pos, tok)


def kernel(pert_index, pert_expression, pos, pert_token):
    return _pert_embed(pert_index.astype(jnp.int32),
                       pert_expression.reshape(BATCH, 1), pos, pert_token)


# speedup vs baseline: 1.5499x; 1.0292x over previous
import functools

import jax
import jax.numpy as jnp
from jax import lax
from jax.experimental import pallas as pl
from jax.experimental.pallas import tpu as pltpu
from jax.experimental.pallas import tpu_sc as plsc

BATCH = 16384
VOCAB = 100000
D = 128
OD = D + 1
NC = 2
NS = 16
NW = NC * NS
ROWS_PER_W = BATCH // NW
CHUNK = 128
NCHUNK = ROWS_PER_W // CHUNK
NBUF = 3
LANES = 16


def _sc_body(idx_hbm, expr_hbm, tab_hbm, tok_hbm, out_hbm,
             idx_v, tok_v,
             rows0, rows1, rows2,
             gsem0, gsem1, gsem2, ssem0, ssem1, ssem2, esem):
    rows = (rows0, rows1, rows2)
    gsem = (gsem0, gsem1, gsem2)
    ssem = (ssem0, ssem1, ssem2)

    wid = lax.axis_index("s") * NC + lax.axis_index("c")
    row0 = wid * ROWS_PER_W
    tab = tab_hbm.at[0]

    pltpu.sync_copy(idx_hbm.at[pl.ds(row0, ROWS_PER_W)], idx_v)
    pltpu.sync_copy(tok_hbm, tok_v)

    ecp = pltpu.async_copy(
        expr_hbm.at[pl.ds(row0, 1)],
        out_hbm.at[pl.ds(row0, 1), 0, pl.ds(D, 1)],
        esem)

    tok = [tok_v[pl.ds(j * LANES, LANES)] for j in range(D // LANES)]

    def compute_chunk(buf):
        @plsc.parallel_loop(0, 1, unroll=1)
        def _(r):
            for j in range(1):
                sl = pl.ds(j * LANES, LANES)
                rows[buf][r, sl] = rows[buf][r, sl] + tok[j]

    def gather(c):
        return pltpu.async_copy(
            tab.at[idx_v.at[pl.ds(c * CHUNK, 8)]], rows[c % NBUF].at[pl.ds(0, 8)],
            gsem[c % NBUF])

    def store(c):
        return pltpu.async_copy(
            rows[c % NBUF].at[pl.ds(0, 8)],
            out_hbm.at[pl.ds(row0 + c * CHUNK, 8), 0, pl.ds(0, D)],
            ssem[c % NBUF])

    g0 = gather(0)
    g1 = gather(1)
    g2 = gather(2)
    g0.wait()
    compute_chunk(0)
    s0 = store(0)
    g1.wait()
    compute_chunk(1)
    s1 = store(1)
    s0.wait()
    g3 = gather(3)
    g2.wait()
    compute_chunk(2)
    s2 = store(2)
    g3.wait()
    compute_chunk(0)
    s3 = store(3)
    s1.wait()
    s2.wait()
    s3.wait()
    ecp.wait()


@jax.jit
def _pert_embed(idx, expr, pos, tok):
    mesh = plsc.VectorSubcoreMesh(core_axis_name="c", subcore_axis_name="s")
    run = functools.partial(
        pl.kernel, mesh=mesh,
        compiler_params=pltpu.CompilerParams(use_tc_tiling_on_sc=False),
        out_type=jax.ShapeDtypeStruct((BATCH, 1, OD), jnp.float32),
        scratch_types=[
            pltpu.VMEM((ROWS_PER_W,), jnp.int32),
            pltpu.VMEM((D,), jnp.float32),
            pltpu.VMEM((CHUNK, D), jnp.float32),
            pltpu.VMEM((CHUNK, D), jnp.float32),
            pltpu.VMEM((CHUNK, D), jnp.float32),
            pltpu.SemaphoreType.DMA,
            pltpu.SemaphoreType.DMA,
            pltpu.SemaphoreType.DMA,
            pltpu.SemaphoreType.DMA,
            pltpu.SemaphoreType.DMA,
            pltpu.SemaphoreType.DMA,
            pltpu.SemaphoreType.DMA,
        ],
    )(_sc_body)
    return run(idx, expr, pos, tok)


def kernel(pert_index, pert_expression, pos, pert_token):
    return _pert_embed(pert_index.astype(jnp.int32),
                       pert_expression.reshape(BATCH, 1), pos, pert_token)


# speedup vs baseline: 2.7330x; 1.7634x over previous
import functools

import jax
import jax.numpy as jnp
from jax import lax
from jax.experimental import pallas as pl
from jax.experimental.pallas import tpu as pltpu
from jax.experimental.pallas import tpu_sc as plsc

BATCH = 16384
VOCAB = 100000
D = 128
OD = D + 1
NC = 2
NS = 16
NW = NC * NS
ROWS_PER_W = BATCH // NW
CHUNK = 128
NCHUNK = ROWS_PER_W // CHUNK
LANES = 16


def _sc_body(idx_hbm, expr_hbm, tab_hbm, tok_hbm, out_hbm,
             idx_v, expr_v, tok_v, tsp_v,
             rows0, rows1, obuf0, obuf1,
             gsem0, gsem1, ssem0, ssem1):
    rows = (rows0, rows1)
    obuf = (obuf0, obuf1)
    gsem = (gsem0, gsem1)
    ssem = (ssem0, ssem1)

    wid = lax.axis_index("s") * NC + lax.axis_index("c")
    row0 = wid * ROWS_PER_W
    tab = tab_hbm.at[0]

    pltpu.sync_copy(idx_hbm.at[pl.ds(row0, ROWS_PER_W)], idx_v)
    pltpu.sync_copy(expr_hbm.at[pl.ds(row0, ROWS_PER_W)], expr_v)
    pltpu.sync_copy(tok_hbm, tok_v)

    for j in range(D // LANES):
        tv = tok_v[pl.ds(j * LANES, LANES)]
        for l in range(LANES):
            tsp_v[pl.ds((j * LANES + l) * LANES, LANES)] = (
                jnp.zeros((LANES,), jnp.float32) + tv[l])

    iota = lax.iota(jnp.int32, LANES)
    rvec = [k * LANES + iota for k in range(CHUNK // LANES)]
    zero = iota * 0

    def compute_chunk(c0, buf):
        @plsc.parallel_loop(0, D, unroll=2)
        def _(c):
            cvec = zero + c
            tv = tsp_v[pl.ds(c * LANES, LANES)]
            for k in range(CHUNK // LANES):
                v = plsc.load_gather(rows[buf], [rvec[k], cvec])
                obuf[buf][c, pl.ds(k * LANES, LANES)] = v + tv
        for k in range(CHUNK // LANES):
            obuf[buf][D, pl.ds(k * LANES, LANES)] = (
                expr_v[pl.ds(c0 * CHUNK + k * LANES, LANES)])

    def gather(c):
        return pltpu.async_copy(
            tab.at[idx_v.at[pl.ds(c * CHUNK, CHUNK)]], rows[c % 2],
            gsem[c % 2])

    def store(c):
        return pltpu.async_copy(
            obuf[c % 2],
            out_hbm.at[pl.ds(0, OD), pl.ds(row0 + c * CHUNK, CHUNK)],
            ssem[c % 2])

    g0 = gather(0)
    g1 = gather(1)
    g0.wait()
    compute_chunk(0, 0)
    s0 = store(0)
    g2 = gather(2)
    g1.wait()
    compute_chunk(1, 1)
    s1 = store(1)
    g3 = gather(3)
    s0.wait()
    g2.wait()
    compute_chunk(2, 0)
    s2 = store(2)
    s1.wait()
    g3.wait()
    compute_chunk(3, 1)
    s3 = store(3)
    s2.wait()
    s3.wait()


@jax.jit
def _pert_embed(idx, expr, pos, tok):
    mesh = plsc.VectorSubcoreMesh(core_axis_name="c", subcore_axis_name="s")
    run = functools.partial(
        pl.kernel, mesh=mesh,
        compiler_params=pltpu.CompilerParams(needs_layout_passes=False),
        out_type=jax.ShapeDtypeStruct((OD, BATCH), jnp.float32),
        scratch_types=[
            pltpu.VMEM((ROWS_PER_W,), jnp.int32),
            pltpu.VMEM((ROWS_PER_W,), jnp.float32),
            pltpu.VMEM((D,), jnp.float32),
            pltpu.VMEM((D * LANES,), jnp.float32),
            pltpu.VMEM((CHUNK, D), jnp.float32),
            pltpu.VMEM((CHUNK, D), jnp.float32),
            pltpu.VMEM((OD, CHUNK), jnp.float32),
            pltpu.VMEM((OD, CHUNK), jnp.float32),
            pltpu.SemaphoreType.DMA,
            pltpu.SemaphoreType.DMA,
            pltpu.SemaphoreType.DMA,
            pltpu.SemaphoreType.DMA,
        ],
    )(_sc_body)
    return run(idx, expr, pos, tok)


def kernel(pert_index, pert_expression, pos, pert_token):
    outT = _pert_embed(pert_index.astype(jnp.int32),
                       pert_expression, pos, pert_token)
    return outT.T.reshape(BATCH, 1, OD)


# speedup vs baseline: 3.3007x; 1.2077x over previous
import functools

import jax
import jax.numpy as jnp
from jax import lax
from jax.experimental import pallas as pl
from jax.experimental.pallas import tpu as pltpu
from jax.experimental.pallas import tpu_sc as plsc

BATCH = 16384
VOCAB = 100000
D = 128
OD = D + 1
NC = 2
NS = 16
NW = NC * NS
ROWS_PER_W = BATCH // NW
CHUNK = 128
NCHUNK = ROWS_PER_W // CHUNK
LANES = 16


def _sc_body(idx_hbm, expr_hbm, tab_hbm, tok_hbm, out_hbm,
             idx_v, expr_v, tok_v, tsp_v,
             rows0, rows1, obuf0, obuf1,
             gsem0, gsem1, ssem0, ssem1):
    rows = (rows0, rows1)
    obuf = (obuf0, obuf1)
    gsem = (gsem0, gsem1)
    ssem = (ssem0, ssem1)

    wid = lax.axis_index("s") * NC + lax.axis_index("c")
    row0 = wid * ROWS_PER_W
    tab = tab_hbm.at[0]

    pltpu.sync_copy(idx_hbm.at[pl.ds(row0, ROWS_PER_W)], idx_v)
    pltpu.sync_copy(expr_hbm.at[pl.ds(row0, ROWS_PER_W)], expr_v)
    pltpu.sync_copy(tok_hbm, tok_v)

    for j in range(D // LANES):
        tv = tok_v[pl.ds(j * LANES, LANES)]
        for l in range(LANES):
            tsp_v[pl.ds((j * LANES + l) * LANES, LANES)] = (
                jnp.zeros((LANES,), jnp.float32) + tv[l])

    iota = lax.iota(jnp.int32, LANES)
    rvec = [k * LANES + iota for k in range(CHUNK // LANES)]
    zero = iota * 0

    def compute_chunk(c0, buf):
        @plsc.parallel_loop(0, D, unroll=2)
        def _(c):
            cvec = zero + c
            tv = tsp_v[pl.ds(c * LANES, LANES)]
            for k in range(CHUNK // LANES):
                v = plsc.load_gather(rows[buf], [rvec[k], cvec])
                obuf[buf][c, 0, pl.ds(k * LANES, LANES)] = v + tv
        for k in range(CHUNK // LANES):
            obuf[buf][D, 0, pl.ds(k * LANES, LANES)] = (
                expr_v[pl.ds(c0 * CHUNK + k * LANES, LANES)])

    def gather(c):
        return pltpu.async_copy(
            tab.at[idx_v.at[pl.ds(c * CHUNK, CHUNK)]], rows[c % 2],
            gsem[c % 2])

    def store(c):
        return pltpu.async_copy(
            obuf[c % 2],
            out_hbm.at[pl.ds(0, OD), pl.ds(wid * NCHUNK + c, 1), pl.ds(0, CHUNK)],
            ssem[c % 2])

    g0 = gather(0)
    g1 = gather(1)
    g0.wait()
    compute_chunk(0, 0)
    s0 = store(0)
    g2 = gather(2)
    g1.wait()
    compute_chunk(1, 1)
    s1 = store(1)
    g3 = gather(3)
    s0.wait()
    g2.wait()
    compute_chunk(2, 0)
    s2 = store(2)
    s1.wait()
    g3.wait()
    compute_chunk(3, 1)
    s3 = store(3)
    s2.wait()
    s3.wait()


@jax.jit
def _pert_embed(idx, expr, pos, tok):
    mesh = plsc.VectorSubcoreMesh(core_axis_name="c", subcore_axis_name="s")
    run = functools.partial(
        pl.kernel, mesh=mesh,
        compiler_params=pltpu.CompilerParams(needs_layout_passes=False),
        out_type=jax.ShapeDtypeStruct((OD, BATCH // CHUNK, CHUNK), jnp.float32),
        scratch_types=[
            pltpu.VMEM((ROWS_PER_W,), jnp.int32),
            pltpu.VMEM((ROWS_PER_W,), jnp.float32),
            pltpu.VMEM((D,), jnp.float32),
            pltpu.VMEM((D * LANES,), jnp.float32),
            pltpu.VMEM((CHUNK, D), jnp.float32),
            pltpu.VMEM((CHUNK, D), jnp.float32),
            pltpu.VMEM((OD, 1, CHUNK), jnp.float32),
            pltpu.VMEM((OD, 1, CHUNK), jnp.float32),
            pltpu.SemaphoreType.DMA,
            pltpu.SemaphoreType.DMA,
            pltpu.SemaphoreType.DMA,
            pltpu.SemaphoreType.DMA,
        ],
    )(_sc_body)
    return run(idx, expr, pos, tok)


def kernel(pert_index, pert_expression, pos, pert_token):
    outT = _pert_embed(pert_index.astype(jnp.int32),
                       pert_expression, pos, pert_token)
    return jnp.transpose(outT, (1, 2, 0)).reshape(BATCH, 1, OD)


# speedup vs baseline: 3.6602x; 1.1089x over previous
import functools

import jax
import jax.numpy as jnp
from jax import lax
from jax.experimental import pallas as pl
from jax.experimental.pallas import tpu as pltpu
from jax.experimental.pallas import tpu_sc as plsc

BATCH = 16384
VOCAB = 100000
D = 128
OD = D + 1
NC = 2
NS = 16
NW = NC * NS
ROWS_PER_W = BATCH // NW
CHUNK = 128
NCHUNK = ROWS_PER_W // CHUNK
LANES = 16


def _sc_body(idx_hbm, expr_hbm, tab_hbm, tok_hbm, out_hbm,
             idx_v, expr_v, tok_v,
             rows0, rows1, obuf0, obuf1,
             gsem0, gsem1, ssem0, ssem1):
    rows = (rows0, rows1)
    obuf = (obuf0, obuf1)
    gsem = (gsem0, gsem1)
    ssem = (ssem0, ssem1)

    wid = lax.axis_index("s") * NC + lax.axis_index("c")
    row0 = wid * ROWS_PER_W
    tab = tab_hbm.at[0]

    pltpu.sync_copy(idx_hbm.at[pl.ds(row0, ROWS_PER_W)], idx_v)
    pltpu.sync_copy(expr_hbm.at[pl.ds(row0, ROWS_PER_W)], expr_v)
    pltpu.sync_copy(tok_hbm, tok_v)

    iota = lax.iota(jnp.int32, LANES)
    rvec = [k * LANES + iota for k in range(CHUNK // LANES)]
    zero = iota * 0

    def compute_chunk(c0, buf):
        @plsc.parallel_loop(0, LANES, unroll=2)
        def _(j):
            bvec = (iota + j) & (LANES - 1)
            for cb in range(D // LANES):
                cvec = bvec + cb * LANES
                tv = plsc.load_gather(tok_v, [cvec])
                for k in range(CHUNK // LANES):
                    v = plsc.load_gather(rows[buf], [rvec[k], cvec])
                    plsc.store_scatter(obuf[buf], [cvec, zero, rvec[k]], v + tv)
        for k in range(CHUNK // LANES):
            obuf[buf][D, 0, pl.ds(k * LANES, LANES)] = (
                expr_v[pl.ds(c0 * CHUNK + k * LANES, LANES)])

    def gather(c):
        return pltpu.async_copy(
            tab.at[idx_v.at[pl.ds(c * CHUNK, CHUNK)]], rows[c % 2],
            gsem[c % 2])

    def store(c):
        return pltpu.async_copy(
            obuf[c % 2],
            out_hbm.at[pl.ds(0, OD), pl.ds(wid * NCHUNK + c, 1), pl.ds(0, CHUNK)],
            ssem[c % 2])

    g0 = gather(0)
    g1 = gather(1)
    g0.wait()
    compute_chunk(0, 0)
    s0 = store(0)
    g2 = gather(2)
    g1.wait()
    compute_chunk(1, 1)
    s1 = store(1)
    g3 = gather(3)
    s0.wait()
    g2.wait()
    compute_chunk(2, 0)
    s2 = store(2)
    s1.wait()
    g3.wait()
    compute_chunk(3, 1)
    s3 = store(3)
    s2.wait()
    s3.wait()


@jax.jit
def _pert_embed(idx, expr, pos, tok):
    mesh = plsc.VectorSubcoreMesh(core_axis_name="c", subcore_axis_name="s")
    run = functools.partial(
        pl.kernel, mesh=mesh,
        compiler_params=pltpu.CompilerParams(needs_layout_passes=False),
        out_type=jax.ShapeDtypeStruct((OD, BATCH // CHUNK, CHUNK), jnp.float32),
        scratch_types=[
            pltpu.VMEM((ROWS_PER_W,), jnp.int32),
            pltpu.VMEM((ROWS_PER_W,), jnp.float32),
            pltpu.VMEM((D,), jnp.float32),
            pltpu.VMEM((CHUNK, D), jnp.float32),
            pltpu.VMEM((CHUNK, D), jnp.float32),
            pltpu.VMEM((OD, 1, CHUNK), jnp.float32),
            pltpu.VMEM((OD, 1, CHUNK), jnp.float32),
            pltpu.SemaphoreType.DMA,
            pltpu.SemaphoreType.DMA,
            pltpu.SemaphoreType.DMA,
            pltpu.SemaphoreType.DMA,
        ],
    )(_sc_body)
    return run(idx, expr, pos, tok)


def kernel(pert_index, pert_expression, pos, pert_token):
    outT = _pert_embed(pert_index.astype(jnp.int32),
                       pert_expression, pos, pert_token)
    return jnp.transpose(outT, (1, 2, 0)).reshape(BATCH, 1, OD)
